# Initial kernel scaffold; baseline (speedup 1.0000x reference)
import jax
import jax.numpy as jnp
from jax.experimental import pallas as pl


def kernel(x, edge_index, W1, b1, Wh, bh, W2, b2):
    raise NotImplementedError("write your pallas kernel here")


# speedup vs baseline: 6.4582x; 6.4582x over previous
import functools

import jax
import jax.numpy as jnp
from jax import lax
from jax.experimental import pallas as pl
from jax.experimental.pallas import tpu as pltpu
from jax.experimental.pallas import tpu_sc as plsc

N = 10000
NP = 10240
E = 160000
NT = 16
C = 128
K = 80
KH = K // 2
EPT = K * C
EP = NT * EPT
RPT = NP // NT
H = 128
F1, FH, FO = 256, 256, 128
R = 512


def _sc_mesh():
    return plsc.VectorSubcoreMesh(core_axis_name="c", subcore_axis_name="s")


def _deg_sc(dstf, zeros_np):
    @functools.partial(
        pl.kernel,
        out_type=jax.ShapeDtypeStruct((NP,), jnp.float32),
        mesh=_sc_mesh(),
        compiler_params=pltpu.CompilerParams(needs_layout_passes=False),
        scratch_types=[
            pltpu.VMEM((EPT,), jnp.int32),
            pltpu.VMEM((NP,), jnp.float32),
            pltpu.VMEM((NT, RPT), jnp.float32),
            pltpu.VMEM((RPT,), jnp.float32),
            pltpu.VMEM_SHARED((NT, NP), jnp.float32),
        ],
    )
    def k(dst_hbm, z_hbm, deg_hbm, dstv, dloc, tmp, tot, spall):
        c = lax.axis_index("c")
        s = lax.axis_index("s")
        pltpu.sync_copy(dst_hbm.at[s], dstv)
        pltpu.sync_copy(z_hbm, dloc)
        ones = jnp.ones((16,), jnp.float32)

        def body(i, carry):
            idx = dstv[pl.ds(i * 16, 16)]
            plsc.addupdate_scatter(dloc, [idx], ones)
            return carry

        lax.fori_loop(0, EPT // 16, body, 0)
        pltpu.sync_copy(dloc, spall.at[s])
        plsc.subcore_barrier()
        pltpu.sync_copy(spall.at[:, pl.ds(s * RPT, RPT)], tmp)

        def mbody(i, carry):
            v = tmp[0, pl.ds(i * 16, 16)]
            for t in range(1, NT):
                v = v + tmp[t, pl.ds(i * 16, 16)]
            tot[pl.ds(i * 16, 16)] = v
            return carry

        lax.fori_loop(0, RPT // 16, mbody, 0)

        @pl.when(c == 0)
        def _():
            pltpu.sync_copy(tot, deg_hbm.at[pl.ds(s * RPT, RPT)])

    return k(dstf, zeros_np)


def _prop_half_sc(init_a, init_b, hh_a, hh_b, src3h, dst3h):
    @functools.partial(
        pl.kernel,
        out_type=(
            jax.ShapeDtypeStruct((NP, H), jnp.float32),
            jax.ShapeDtypeStruct((NP, H), jnp.float32),
        ),
        mesh=_sc_mesh(),
        compiler_params=pltpu.CompilerParams(needs_layout_passes=False),
        scratch_types=[
            pltpu.VMEM((KH, C), jnp.int32),
            pltpu.VMEM((KH, C), jnp.int32),
            pltpu.VMEM((2, C, H), jnp.float32),
            pltpu.SemaphoreType.DMA((2,)),
            pltpu.VMEM_SHARED((NP, H), jnp.float32),
        ],
    )
    def k(ia, ib, ha, hb, s3, d3, oa, ob, srcv, dstv, buf, sems, acc):
        c = lax.axis_index("c")
        s = lax.axis_index("s")
        pltpu.sync_copy(s3.at[s], srcv)
        pltpu.sync_copy(d3.at[s], dstv)
        r0 = s * RPT

        def run(init, hh, out):
            pltpu.sync_copy(init.at[pl.ds(r0, RPT)], acc.at[pl.ds(r0, RPT)])
            plsc.subcore_barrier()
            pltpu.async_copy(hh.at[srcv.at[0]], buf.at[0], sems.at[0])
            pltpu.async_copy(hh.at[srcv.at[1]], buf.at[1], sems.at[1])

            def body(i, carry):
                for b in range(2):
                    kk = i * 2 + b
                    pltpu.make_async_copy(
                        hh.at[srcv.at[kk]], buf.at[b], sems.at[b]
                    ).wait()
                    pltpu.sync_copy(buf.at[b], acc.at[dstv.at[kk]], add=True)
                    nk = kk + 2

                    @pl.when(nk < KH)
                    def _():
                        pltpu.async_copy(hh.at[srcv.at[nk]], buf.at[b], sems.at[b])

                return carry

            lax.fori_loop(0, KH // 2, body, 0)
            plsc.subcore_barrier()
            pltpu.sync_copy(acc.at[pl.ds(r0, RPT)], out.at[pl.ds(r0, RPT)])

        @pl.when(c == 0)
        def _():
            run(ia, ha, oa)

        @pl.when(c == 1)
        def _():
            run(ib, hb, ob)

    return k(init_a, init_b, hh_a, hh_b, src3h, dst3h)


def _prop_sc(hh_a, hh_b, idx):
    s3a, d3a, s3b, d3b = idx
    p_a, p_b = _prop_half_sc(hh_a, hh_b, hh_a, hh_b, s3a, d3a)
    return _prop_half_sc(p_a, p_b, hh_a, hh_b, s3b, d3b)


def _tc_first(xp, deg2, W):
    def body(x_ref, deg_ref, w_ref, oa_ref, ob_ref):
        dinv = lax.rsqrt(deg_ref[...] + 1.0)
        m = jnp.dot(
            x_ref[...], w_ref[...],
            preferred_element_type=jnp.float32,
            precision=lax.Precision.HIGHEST,
        )
        hh = m * dinv
        oa_ref[...] = hh[:, :H]
        ob_ref[...] = hh[:, H:]

    return pl.pallas_call(
        body,
        grid=(NP // R,),
        in_specs=[
            pl.BlockSpec((R, F1), lambda i: (i, 0)),
            pl.BlockSpec((R, 1), lambda i: (i, 0)),
            pl.BlockSpec((F1, FH), lambda i: (0, 0)),
        ],
        out_specs=[
            pl.BlockSpec((R, H), lambda i: (i, 0)),
            pl.BlockSpec((R, H), lambda i: (i, 0)),
        ],
        out_shape=[jax.ShapeDtypeStruct((NP, H), jnp.float32)] * 2,
    )(xp, deg2, W)


def _tc_mid(aa, ab, deg2, bias_prev, W, Dout):
    Din = FH

    def body(aa_ref, ab_ref, deg_ref, b_ref, w_ref, oa_ref, ob_ref):
        dinv = lax.rsqrt(deg_ref[...] + 1.0)
        z = jnp.concatenate([aa_ref[...], ab_ref[...]], axis=1) * dinv + b_ref[...]
        t = jnp.maximum(z, 0.0)
        hh = jnp.dot(
            t, w_ref[...],
            preferred_element_type=jnp.float32,
            precision=lax.Precision.HIGHEST,
        ) * dinv
        if Dout < 2 * H:
            pad = jnp.zeros((R, H - Dout // 2), jnp.float32)
            oa_ref[...] = jnp.concatenate([hh[:, : Dout // 2], pad], axis=1)
            ob_ref[...] = jnp.concatenate([hh[:, Dout // 2 :], pad], axis=1)
        else:
            oa_ref[...] = hh[:, :H]
            ob_ref[...] = hh[:, H:]

    return pl.pallas_call(
        body,
        grid=(NP // R,),
        in_specs=[
            pl.BlockSpec((R, H), lambda i: (i, 0)),
            pl.BlockSpec((R, H), lambda i: (i, 0)),
            pl.BlockSpec((R, 1), lambda i: (i, 0)),
            pl.BlockSpec((1, Din), lambda i: (0, 0)),
            pl.BlockSpec((Din, Dout), lambda i: (0, 0)),
        ],
        out_specs=[
            pl.BlockSpec((R, H), lambda i: (i, 0)),
            pl.BlockSpec((R, H), lambda i: (i, 0)),
        ],
        out_shape=[jax.ShapeDtypeStruct((NP, H), jnp.float32)] * 2,
    )(aa, ab, deg2, bias_prev, W)


def _tc_final(aa, ab, deg2, bias):
    def body(aa_ref, ab_ref, deg_ref, b_ref, o_ref):
        dinv = lax.rsqrt(deg_ref[...] + 1.0)
        z = (
            jnp.concatenate(
                [aa_ref[...][:, : FO // 2], ab_ref[...][:, : FO // 2]], axis=1
            )
            * dinv
            + b_ref[...]
        )
        m = jnp.max(z, axis=1, keepdims=True)
        e = jnp.exp(z - m)
        ssum = jnp.sum(e, axis=1, keepdims=True)
        o_ref[...] = z - m - jnp.log(ssum)

    return pl.pallas_call(
        body,
        grid=(NP // R,),
        in_specs=[
            pl.BlockSpec((R, H), lambda i: (i, 0)),
            pl.BlockSpec((R, H), lambda i: (i, 0)),
            pl.BlockSpec((R, 1), lambda i: (i, 0)),
            pl.BlockSpec((1, FO), lambda i: (0, 0)),
        ],
        out_specs=pl.BlockSpec((R, FO), lambda i: (i, 0)),
        out_shape=jax.ShapeDtypeStruct((N, FO), jnp.float32),
    )(aa, ab, deg2, bias)


def kernel(x, edge_index, W1, b1, Wh, bh, W2, b2):
    ei = edge_index.astype(jnp.int32)
    padv = jnp.full((EP - E,), N, jnp.int32)
    src = jnp.concatenate([ei[0], padv])
    dst = jnp.concatenate([ei[1], padv])
    src3 = src.reshape(NT, K, C)
    dst3 = dst.reshape(NT, K, C)
    idx = (
        src3[:, :KH],
        dst3[:, :KH],
        src3[:, KH:],
        dst3[:, KH:],
    )
    dstf = dst.reshape(NT, EPT)
    xp = jnp.pad(x, ((0, NP - N), (0, 0)))
    zeros_np = jnp.zeros((NP,), jnp.float32)

    deg = _deg_sc(dstf, zeros_np)
    deg2 = deg.reshape(NP, 1)

    hh_a, hh_b = _tc_first(xp, deg2, W1)
    a1, a1b = _prop_sc(hh_a, hh_b, idx)
    hh_a, hh_b = _tc_mid(a1, a1b, deg2, b1.reshape(1, FH), Wh, FH)
    a2, a2b = _prop_sc(hh_a, hh_b, idx)
    hh_a, hh_b = _tc_mid(a2, a2b, deg2, bh.reshape(1, FH), W2, FO)
    a3, a3b = _prop_sc(hh_a, hh_b, idx)
    return _tc_final(a3, a3b, deg2, b2.reshape(1, FO))


# speedup vs baseline: 13.7212x; 2.1246x over previous
import functools

import jax
import jax.numpy as jnp
from jax import lax
from jax.experimental import pallas as pl
from jax.experimental.pallas import tpu as pltpu
from jax.experimental.pallas import tpu_sc as plsc

N = 10000
NP = 10240
E = 160000
NT = 16
C = 128
K = 80
KH = K // 2
EPT = K * C
EP = NT * EPT
RPT = NP // NT
H = 128
F1, FH, FO = 256, 256, 128
R = 512


def _sc_mesh():
    return plsc.VectorSubcoreMesh(core_axis_name="c", subcore_axis_name="s")


def _deg_sc(dstf, zeros_np):
    @functools.partial(
        pl.kernel,
        out_type=jax.ShapeDtypeStruct((NP,), jnp.float32),
        mesh=_sc_mesh(),
        compiler_params=pltpu.CompilerParams(needs_layout_passes=False),
        scratch_types=[
            pltpu.VMEM((EPT,), jnp.int32),
            pltpu.VMEM((NP,), jnp.float32),
            pltpu.VMEM((NT, RPT), jnp.float32),
            pltpu.VMEM((RPT,), jnp.float32),
            pltpu.VMEM_SHARED((NT, NP), jnp.float32),
        ],
    )
    def k(dst_hbm, z_hbm, deg_hbm, dstv, dloc, tmp, tot, spall):
        c = lax.axis_index("c")
        s = lax.axis_index("s")
        pltpu.sync_copy(dst_hbm.at[s], dstv)
        pltpu.sync_copy(z_hbm, dloc)
        ones = jnp.ones((16,), jnp.float32)

        def body(i, carry):
            idx = dstv[pl.ds(i * 16, 16)]
            plsc.addupdate_scatter(dloc, [idx], ones)
            return carry

        lax.fori_loop(0, EPT // 16, body, 0)
        pltpu.sync_copy(dloc, spall.at[s])
        plsc.subcore_barrier()
        pltpu.sync_copy(spall.at[:, pl.ds(s * RPT, RPT)], tmp)

        def mbody(i, carry):
            v = tmp[0, pl.ds(i * 16, 16)]
            for t in range(1, NT):
                v = v + tmp[t, pl.ds(i * 16, 16)]
            tot[pl.ds(i * 16, 16)] = v
            return carry

        lax.fori_loop(0, RPT // 16, mbody, 0)

        @pl.when(c == 0)
        def _():
            pltpu.sync_copy(tot, deg_hbm.at[pl.ds(s * RPT, RPT)])

    return k(dstf, zeros_np)


def _prop_half_sc(init_a, init_b, hh_a, hh_b, src3h, dst3h):
    @functools.partial(
        pl.kernel,
        out_type=(
            jax.ShapeDtypeStruct((NP, H), jnp.float32),
            jax.ShapeDtypeStruct((NP, H), jnp.float32),
        ),
        mesh=_sc_mesh(),
        compiler_params=pltpu.CompilerParams(needs_layout_passes=False),
        scratch_types=[
            pltpu.VMEM((KH, C), jnp.int32),
            pltpu.VMEM((KH, C), jnp.int32),
            pltpu.VMEM((2, C, H), jnp.float32),
            pltpu.SemaphoreType.DMA((2,)),
            pltpu.VMEM_SHARED((NP, H), jnp.float32),
        ],
    )
    def k(ia, ib, ha, hb, s3, d3, oa, ob, srcv, dstv, buf, sems, acc):
        c = lax.axis_index("c")
        s = lax.axis_index("s")
        pltpu.sync_copy(s3.at[s], srcv)
        pltpu.sync_copy(d3.at[s], dstv)
        r0 = s * RPT

        def run(init, hh, out):
            pltpu.sync_copy(init.at[pl.ds(r0, RPT)], acc.at[pl.ds(r0, RPT)])
            plsc.subcore_barrier()
            pltpu.async_copy(hh.at[srcv.at[0]], buf.at[0], sems.at[0])
            pltpu.async_copy(hh.at[srcv.at[1]], buf.at[1], sems.at[1])

            def body(i, carry):
                for b in range(2):
                    kk = i * 2 + b
                    pltpu.make_async_copy(
                        hh.at[srcv.at[kk]], buf.at[b], sems.at[b]
                    ).wait()
                    pltpu.sync_copy(buf.at[b], acc.at[dstv.at[kk]], add=True)
                    nk = kk + 2

                    @pl.when(nk < KH)
                    def _():
                        pltpu.async_copy(hh.at[srcv.at[nk]], buf.at[b], sems.at[b])

                return carry

            lax.fori_loop(0, KH // 2, body, 0)
            plsc.subcore_barrier()
            pltpu.sync_copy(acc.at[pl.ds(r0, RPT)], out.at[pl.ds(r0, RPT)])

        @pl.when(c == 0)
        def _():
            run(ia, ha, oa)

        @pl.when(c == 1)
        def _():
            run(ib, hb, ob)

    return k(init_a, init_b, hh_a, hh_b, src3h, dst3h)


def _prop_sc(hh_a, hh_b, idx):
    s3a, d3a, s3b, d3b = idx
    p_a, p_b = _prop_half_sc(hh_a, hh_b, hh_a, hh_b, s3a, d3a)
    return _prop_half_sc(p_a, p_b, hh_a, hh_b, s3b, d3b)


def _tc_first(xp, deg2, W):
    def body(x_ref, deg_ref, w_ref, oa_ref, ob_ref):
        dinv = lax.rsqrt(deg_ref[...] + 1.0)
        m = jnp.dot(
            x_ref[...], w_ref[...],
            preferred_element_type=jnp.float32,
            precision=lax.Precision.HIGHEST,
        )
        hh = m * dinv
        oa_ref[...] = hh[:, :H]
        ob_ref[...] = hh[:, H:]

    return pl.pallas_call(
        body,
        grid=(NP // R,),
        in_specs=[
            pl.BlockSpec((R, F1), lambda i: (i, 0)),
            pl.BlockSpec((R, 1), lambda i: (i, 0)),
            pl.BlockSpec((F1, FH), lambda i: (0, 0)),
        ],
        out_specs=[
            pl.BlockSpec((R, H), lambda i: (i, 0)),
            pl.BlockSpec((R, H), lambda i: (i, 0)),
        ],
        out_shape=[jax.ShapeDtypeStruct((NP, H), jnp.float32)] * 2,
    )(xp, deg2, W)


def _tc_mid(aa, ab, deg2, bias_prev, W, Dout):
    Din = FH

    def body(aa_ref, ab_ref, deg_ref, b_ref, w_ref, oa_ref, ob_ref):
        dinv = lax.rsqrt(deg_ref[...] + 1.0)
        z = jnp.concatenate([aa_ref[...], ab_ref[...]], axis=1) * dinv + b_ref[...]
        t = jnp.maximum(z, 0.0)
        hh = jnp.dot(
            t, w_ref[...],
            preferred_element_type=jnp.float32,
            precision=lax.Precision.HIGHEST,
        ) * dinv
        if Dout < 2 * H:
            pad = jnp.zeros((R, H - Dout // 2), jnp.float32)
            oa_ref[...] = jnp.concatenate([hh[:, : Dout // 2], pad], axis=1)
            ob_ref[...] = jnp.concatenate([hh[:, Dout // 2 :], pad], axis=1)
        else:
            oa_ref[...] = hh[:, :H]
            ob_ref[...] = hh[:, H:]

    return pl.pallas_call(
        body,
        grid=(NP // R,),
        in_specs=[
            pl.BlockSpec((R, H), lambda i: (i, 0)),
            pl.BlockSpec((R, H), lambda i: (i, 0)),
            pl.BlockSpec((R, 1), lambda i: (i, 0)),
            pl.BlockSpec((1, Din), lambda i: (0, 0)),
            pl.BlockSpec((Din, Dout), lambda i: (0, 0)),
        ],
        out_specs=[
            pl.BlockSpec((R, H), lambda i: (i, 0)),
            pl.BlockSpec((R, H), lambda i: (i, 0)),
        ],
        out_shape=[jax.ShapeDtypeStruct((NP, H), jnp.float32)] * 2,
    )(aa, ab, deg2, bias_prev, W)


def _tc_final(aa, ab, deg2, bias):
    def body(aa_ref, ab_ref, deg_ref, b_ref, o_ref):
        dinv = lax.rsqrt(deg_ref[...] + 1.0)
        z = (
            jnp.concatenate(
                [aa_ref[...][:, : FO // 2], ab_ref[...][:, : FO // 2]], axis=1
            )
            * dinv
            + b_ref[...]
        )
        m = jnp.max(z, axis=1, keepdims=True)
        e = jnp.exp(z - m)
        ssum = jnp.sum(e, axis=1, keepdims=True)
        o_ref[...] = z - m - jnp.log(ssum)

    return pl.pallas_call(
        body,
        grid=(NP // R,),
        in_specs=[
            pl.BlockSpec((R, H), lambda i: (i, 0)),
            pl.BlockSpec((R, H), lambda i: (i, 0)),
            pl.BlockSpec((R, 1), lambda i: (i, 0)),
            pl.BlockSpec((1, FO), lambda i: (0, 0)),
        ],
        out_specs=pl.BlockSpec((R, FO), lambda i: (i, 0)),
        out_shape=jax.ShapeDtypeStruct((N, FO), jnp.float32),
    )(aa, ab, deg2, bias)


def kernel(x, edge_index, W1, b1, Wh, bh, W2, b2):
    ei = edge_index.astype(jnp.int32)
    padv = N + (jnp.arange(EP - E, dtype=jnp.int32) % (NP - N))
    src = jnp.concatenate([ei[0], padv])
    dst = jnp.concatenate([ei[1], padv])
    src3 = src.reshape(NT, K, C)
    dst3 = dst.reshape(NT, K, C)
    idx = (
        src3[:, :KH],
        dst3[:, :KH],
        src3[:, KH:],
        dst3[:, KH:],
    )
    dstf = dst.reshape(NT, EPT)
    xp = jnp.pad(x, ((0, NP - N), (0, 0)))
    zeros_np = jnp.zeros((NP,), jnp.float32)

    deg = _deg_sc(dstf, zeros_np)
    deg2 = deg.reshape(NP, 1)

    hh_a, hh_b = _tc_first(xp, deg2, W1)
    a1, a1b = _prop_sc(hh_a, hh_b, idx)
    hh_a, hh_b = _tc_mid(a1, a1b, deg2, b1.reshape(1, FH), Wh, FH)
    a2, a2b = _prop_sc(hh_a, hh_b, idx)
    hh_a, hh_b = _tc_mid(a2, a2b, deg2, bh.reshape(1, FH), W2, FO)
    a3, a3b = _prop_sc(hh_a, hh_b, idx)
    return _tc_final(a3, a3b, deg2, b2.reshape(1, FO))
